# Initial kernel scaffold; baseline (speedup 1.0000x reference)
import jax
import jax.numpy as jnp
from jax.experimental import pallas as pl


def kernel(paper_feat, author_feat, wb_src, wb_dst, wb_eweight, pi_src, pi_dst, pi_ealpha, pi_eweight, hp_src, hp_dst, hp_feat, hp_ealpha):
    raise NotImplementedError("write your pallas kernel here")


# speedup vs baseline: 1.6885x; 1.6885x over previous
import functools

import jax
import jax.numpy as jnp
from jax import lax
from jax.experimental import pallas as pl
from jax.experimental.pallas import tpu as pltpu
from jax.experimental.pallas import tpu_sc as plsc

N_PAPER = 100000
N_AUTHOR = 50000
N_VENUE = 10000
D = 128
L = 16
NS = 16
NC = 2

K = 128
ACHUNK = 12544
ZB = 80
ACC_ROWS = 12800

WB_E0, PI_E0, HP_E0 = 300000, 100000, 200000
WB_BLKS = 2352
PI_BLKS = 800
HP_BLKS = 1568
WB_E = WB_BLKS * K
PI_E = PI_BLKS * K
HP_E = HP_BLKS * K

_mesh = plsc.VectorSubcoreMesh(
    core_axis_name="c", subcore_axis_name="s", num_cores=NC, num_subcores=NS)


def _sc_body(paper, author, wb_src, wb_dst, wb_w,
             pi_src, pi_dst, pi_a, pi_w, hp_src, hp_dst, hp_f, hp_a,
             author_out, vp0, vp1,
             acc, zbuf, rows, src_i, dstl, wv, av):
  c = lax.axis_index("c")
  s = lax.axis_index("s")
  wid = c * NS + s

  @pl.loop(0, ZB)
  def _(r):
    for j in range(D // L):
      zbuf[r, pl.ds(j * L, L)] = jnp.zeros((L,), jnp.float32)

  def zero_acc():
    rows_per_tile = ACC_ROWS // NS
    for i in range(rows_per_tile // ZB):
      pltpu.sync_copy(zbuf, acc.at[pl.ds(s * rows_per_tile + i * ZB, ZB)])
    plsc.subcore_barrier()

  def edge_pass(n_blocks, first_blk, src_ref, dst_ref, w_ref, feat_ref,
                chunk_base, chunk_size, trash, w2_ref=None):
    @pl.loop(0, n_blocks)
    def _(b):
      base = (first_blk + b) * K
      pltpu.sync_copy(src_ref.at[pl.ds(base, K)], src_i)
      pltpu.sync_copy(dst_ref.at[pl.ds(base, K)], dstl)
      pltpu.sync_copy(w_ref.at[pl.ds(base, K)], wv)
      if w2_ref is not None:
        pltpu.sync_copy(w2_ref.at[pl.ds(base, K)], av)
      for i in range(K // L):
        sl = pl.ds(i * L, L)
        d16 = dstl[sl]
        msk = (d16 >= chunk_base) & (d16 < chunk_base + chunk_size)
        dstl[sl] = jnp.where(msk, d16 - chunk_base, trash)
      pltpu.sync_copy(feat_ref.at[src_i], rows)

      @pl.loop(0, K // L)
      def _(g):
        w16 = wv[pl.ds(g * L, L)]
        if w2_ref is not None:
          w16 = w16 * av[pl.ds(g * L, L)]
        for lane in range(L):
          r = g * L + lane
          w = w16[lane]
          for j in range(D // L):
            sl = pl.ds(j * L, L)
            rows[r, sl] = rows[r, sl] * w

      pltpu.sync_copy(rows, acc.at[dstl], add=True)

  vrows = 1000
  zero_acc()
  edge_pass(PI_BLKS // (NC * NS), c * (PI_BLKS // NC) + s * (PI_BLKS // (NC * NS)),
            pi_src, pi_dst, pi_a, paper,
            0, N_VENUE, 0, w2_ref=pi_w)
  edge_pass(HP_BLKS // (NC * NS), c * (HP_BLKS // NC) + s * (HP_BLKS // (NC * NS)),
            hp_src, hp_dst, hp_f, author,
            0, N_VENUE, 0, w2_ref=hp_a)
  plsc.subcore_barrier()
  vb = s * vrows

  @pl.when((c == 0) & (s < N_VENUE // vrows))
  def _():
    pltpu.sync_copy(acc.at[pl.ds(vb, vrows)], vp0.at[pl.ds(vb, vrows)])

  @pl.when((c == 1) & (s < N_VENUE // vrows))
  def _():
    pltpu.sync_copy(acc.at[pl.ds(vb, vrows)], vp1.at[pl.ds(vb, vrows)])
  plsc.subcore_barrier()

  per_tile = ACHUNK // NS
  for chunk in range(4):
    owner = chunk // 2
    crow = chunk * ACHUNK

    @pl.when(c == owner)
    def _():
      zero_acc()
      edge_pass(WB_BLKS // NS, s * (WB_BLKS // NS),
                wb_src, wb_dst, wb_w, paper,
                crow, ACHUNK, ACHUNK)
      plsc.subcore_barrier()
      wbase = s * per_tile
      pltpu.sync_copy(acc.at[pl.ds(wbase, per_tile)],
                      author_out.at[pl.ds(crow + wbase, per_tile)])
      plsc.subcore_barrier()


_sc_kernel = functools.partial(
    pl.kernel,
    out_type=[
        jax.ShapeDtypeStruct((4 * ACHUNK, D), jnp.float32),
        jax.ShapeDtypeStruct((N_VENUE, D), jnp.float32),
        jax.ShapeDtypeStruct((N_VENUE, D), jnp.float32),
    ],
    mesh=_mesh,
    scratch_types=[
        pltpu.VMEM_SHARED((ACC_ROWS, D), jnp.float32),
        pltpu.VMEM((ZB, D), jnp.float32),
        pltpu.VMEM((K, D), jnp.float32),
        pltpu.VMEM((K,), jnp.int32),
        pltpu.VMEM((K,), jnp.int32),
        pltpu.VMEM((K,), jnp.float32),
        pltpu.VMEM((K,), jnp.float32),
    ],
)(_sc_body)


def _add_body(a_ref, b_ref, o_ref):
  o_ref[...] = a_ref[...] + b_ref[...]


def _combine(a, b):
  return pl.pallas_call(
      _add_body,
      out_shape=jax.ShapeDtypeStruct((N_VENUE, D), jnp.float32),
      grid=(10,),
      in_specs=[pl.BlockSpec((N_VENUE // 10, D), lambda i: (i, 0))] * 2,
      out_specs=pl.BlockSpec((N_VENUE // 10, D), lambda i: (i, 0)),
  )(a, b)


def _pad(x, n):
  return jnp.pad(x.reshape(-1), (0, n - x.shape[0]))


def kernel(paper_feat, author_feat, wb_src, wb_dst, wb_eweight,
           pi_src, pi_dst, pi_ealpha, pi_eweight,
           hp_src, hp_dst, hp_feat, hp_ealpha):
  wb_s = _pad(wb_src, WB_E)
  wb_d = _pad(wb_dst, WB_E)
  wb_w = _pad(wb_eweight, WB_E)
  pi_s = _pad(pi_src, PI_E)
  pi_d = _pad(pi_dst, PI_E)
  pi_a = _pad(pi_ealpha, PI_E)
  pi_w = _pad(pi_eweight, PI_E)
  hp_s = _pad(hp_src, HP_E)
  hp_d = _pad(hp_dst, HP_E)
  hp_f = _pad(hp_feat, HP_E)
  hp_a = _pad(hp_ealpha, HP_E)

  author_pad, vp0, vp1 = _sc_kernel(
      paper_feat, author_feat, wb_s, wb_d, wb_w,
      pi_s, pi_d, pi_a, pi_w, hp_s, hp_d, hp_f, hp_a)
  venue_h = _combine(vp0, vp1)
  return (paper_feat, author_pad[:N_AUTHOR], venue_h)


# speedup vs baseline: 2.5640x; 1.5185x over previous
import functools

import jax
import jax.numpy as jnp
from jax import lax
from jax.experimental import pallas as pl
from jax.experimental.pallas import tpu as pltpu
from jax.experimental.pallas import tpu_sc as plsc

N_PAPER = 100000
N_AUTHOR = 50000
N_VENUE = 10000
D = 128
L = 16
NS = 16
NC = 2

K = 128
ACHUNK = 10112
NCHUNK = 5
ZB = 40
ACC_ROWS = 10240

WB_BLKS = 2352
PI_BLKS = 800
HP_BLKS = 1568
WB_E = WB_BLKS * K
PI_E = PI_BLKS * K
HP_E = HP_BLKS * K

_mesh = plsc.VectorSubcoreMesh(
    core_axis_name="c", subcore_axis_name="s", num_cores=NC, num_subcores=NS)


def _sc_body(paper, author, wb_pack, pi_pack, hp_pack,
             author_out, vp0, vp1,
             acc, zbuf, rows2, pbufv, pbufa,
             stage_s, stage_d, stage_w, fsrc, fdst, isem, gsem):
  c = lax.axis_index("c")
  s = lax.axis_index("s")

  @pl.loop(0, ZB)
  def _(r):
    for j in range(D // L):
      zbuf[r, pl.ds(j * L, L)] = jnp.zeros((L,), jnp.float32)

  def zero_acc():
    rows_per_tile = ACC_ROWS // NS
    for i in range(rows_per_tile // ZB):
      pltpu.sync_copy(zbuf, acc.at[pl.ds(s * rows_per_tile + i * ZB, ZB)])
    plsc.subcore_barrier()

  def scale_rows(base, w_from):
    @pl.loop(0, K // L)
    def _(g):
      w16 = w_from(g)
      for lane in range(L):
        r = base + g * L + lane
        w = w16[lane]
        for j in range(D // L):
          sl = pl.ds(j * L, L)
          rows2[r, sl] = rows2[r, sl] * w

  def venue_pass(n_blocks, first_blk, pack_ref, feat_ref):
    def idx_issue(i):
      pltpu.async_copy(pack_ref.at[first_blk + i],
                       pbufv.at[pl.ds(lax.rem(i, 3) * 4, 4)], isem)

    def idx_wait():
      pltpu.make_async_copy(pack_ref.at[0], pbufv.at[pl.ds(0, 4)],
                            isem).wait()

    def gather_issue(i, buf):
      pltpu.async_copy(feat_ref.at[pbufv.at[lax.rem(i, 3) * 4]],
                       rows2.at[pl.ds(buf * K, K)], gsem)

    def gather_wait(buf):
      pltpu.make_async_copy(feat_ref.at[pl.ds(0, K)],
                            rows2.at[pl.ds(buf * K, K)], gsem).wait()

    idx_issue(0)
    idx_wait()
    gather_issue(0, 0)
    if n_blocks > 1:
      idx_issue(1)

    @pl.loop(0, n_blocks)
    def _(i):
      cur = lax.rem(i, 2)
      slot = lax.rem(i, 3)
      gather_wait(cur)

      @pl.when(i + 1 < n_blocks)
      def _():
        idx_wait()
        gather_issue(i + 1, 1 - cur)

      @pl.when(i + 2 < n_blocks)
      def _():
        idx_issue(i + 2)

      def wv16(g):
        a = plsc.bitcast(pbufv[slot * 4 + 2, pl.ds(g * L, L)], jnp.float32)
        b = plsc.bitcast(pbufv[slot * 4 + 3, pl.ds(g * L, L)], jnp.float32)
        return a * b
      scale_rows(cur * K, wv16)
      pltpu.sync_copy(rows2.at[pl.ds(cur * K, K)],
                      acc.at[pbufv.at[slot * 4 + 1]], add=True)

  def flush_stage(feat_ref):
    for g in range(K // L):
      sl = pl.ds(g * L, L)
      fsrc[sl] = stage_s[sl]
      fdst[sl] = stage_d[sl]
    pltpu.sync_copy(feat_ref.at[fsrc], rows2.at[pl.ds(0, K)])
    scale_rows(0, lambda g: stage_w[pl.ds(g * L, L)])
    pltpu.sync_copy(rows2.at[pl.ds(0, K)], acc.at[fdst], add=True)
    for g in range(K // L):
      sl = pl.ds(g * L, L)
      s2 = pl.ds(K + g * L, L)
      stage_s[sl] = stage_s[s2]
      stage_d[sl] = stage_d[s2]
      stage_w[sl] = stage_w[s2]

  def compact_pass(n_blocks, first_blk, pack_ref, feat_ref, chunk_base,
                   trash):
    def idx_issue(i):
      pltpu.async_copy(pack_ref.at[first_blk + i],
                       pbufa.at[pl.ds(lax.rem(i, 2) * 3, 3)], isem)

    def idx_wait():
      pltpu.make_async_copy(pack_ref.at[0], pbufa.at[pl.ds(0, 3)],
                            isem).wait()

    idx_issue(0)

    @pl.loop(0, n_blocks, init_carry=jnp.int32(0))
    def fill(b, fill):
      slot = lax.rem(b, 2)
      idx_wait()

      @pl.when(b + 1 < n_blocks)
      def _():
        idx_issue(b + 1)

      for g in range(K // L):
        sl = pl.ds(g * L, L)
        s16 = pbufa[slot * 3, sl]
        d16 = pbufa[slot * 3 + 1, sl]
        w16 = plsc.bitcast(pbufa[slot * 3 + 2, sl], jnp.float32)
        msk = (d16 >= chunk_base) & (d16 < chunk_base + ACHUNK)
        sdst, ssrc, _ = plsc.sort_key_val(d16 - chunk_base, s16, mask=msk)
        _, sw, _ = plsc.sort_key_val(d16 - chunk_base, w16, mask=msk)
        stage_s[pl.ds(fill, L)] = ssrc
        stage_d[pl.ds(fill, L)] = sdst
        stage_w[pl.ds(fill, L)] = sw
        cnt = plsc.all_reduce_population_count(msk)
        if getattr(cnt, "ndim", 0):
          cnt = cnt[0]
        fill = fill + cnt
      do = fill >= K
      pl.when(do)(lambda: flush_stage(feat_ref))
      return jnp.where(do, fill - K, fill)

    for g in range(K // L):
      sl = pl.ds(g * L, L)
      pos = lax.iota(jnp.int32, L) + g * L
      valid = pos < fill
      stage_d[sl] = jnp.where(valid, stage_d[sl], trash)
      stage_s[sl] = jnp.where(valid, stage_s[sl], 0)
    flush_stage(feat_ref)

  vrows = 1000
  zero_acc()
  venue_pass(PI_BLKS // (NC * NS),
             c * (PI_BLKS // NC) + s * (PI_BLKS // (NC * NS)), pi_pack, paper)
  venue_pass(HP_BLKS // (NC * NS),
             c * (HP_BLKS // NC) + s * (HP_BLKS // (NC * NS)), hp_pack,
             author)
  plsc.subcore_barrier()
  vb = s * vrows

  @pl.when((c == 0) & (s < N_VENUE // vrows))
  def _():
    pltpu.sync_copy(acc.at[pl.ds(vb, vrows)], vp0.at[pl.ds(vb, vrows)])

  @pl.when((c == 1) & (s < N_VENUE // vrows))
  def _():
    pltpu.sync_copy(acc.at[pl.ds(vb, vrows)], vp1.at[pl.ds(vb, vrows)])
  plsc.subcore_barrier()

  per_tile = ACHUNK // NS
  for chunk in range(NCHUNK):
    owner = 0 if chunk < 3 else 1
    crow = chunk * ACHUNK

    @pl.when(c == owner)
    def _():
      zero_acc()
      compact_pass(WB_BLKS // NS, s * (WB_BLKS // NS),
                   wb_pack, paper, crow, ACHUNK)
      plsc.subcore_barrier()
      wbase = s * per_tile
      pltpu.sync_copy(acc.at[pl.ds(wbase, per_tile)],
                      author_out.at[pl.ds(crow + wbase, per_tile)])
      plsc.subcore_barrier()


_sc_kernel = functools.partial(
    pl.kernel,
    out_type=[
        jax.ShapeDtypeStruct((NCHUNK * ACHUNK, D), jnp.float32),
        jax.ShapeDtypeStruct((N_VENUE, D), jnp.float32),
        jax.ShapeDtypeStruct((N_VENUE, D), jnp.float32),
    ],
    mesh=_mesh,
    compiler_params=pltpu.CompilerParams(needs_layout_passes=False),
    scratch_types=[
        pltpu.VMEM_SHARED((ACC_ROWS, D), jnp.float32),
        pltpu.VMEM((ZB, D), jnp.float32),
        pltpu.VMEM((2 * K, D), jnp.float32),
        pltpu.VMEM((12, K), jnp.int32),
        pltpu.VMEM((6, K), jnp.int32),
        pltpu.VMEM((2 * K,), jnp.int32),
        pltpu.VMEM((2 * K,), jnp.int32),
        pltpu.VMEM((2 * K,), jnp.float32),
        pltpu.VMEM((K,), jnp.int32),
        pltpu.VMEM((K,), jnp.int32),
        pltpu.SemaphoreType.DMA,
        pltpu.SemaphoreType.DMA,
    ],
)(_sc_body)


def _add_body(a_ref, b_ref, o_ref):
  o_ref[...] = a_ref[...] + b_ref[...]


def _combine(a, b):
  return pl.pallas_call(
      _add_body,
      out_shape=jax.ShapeDtypeStruct((N_VENUE, D), jnp.float32),
      grid=(10,),
      in_specs=[pl.BlockSpec((N_VENUE // 10, D), lambda i: (i, 0))] * 2,
      out_specs=pl.BlockSpec((N_VENUE // 10, D), lambda i: (i, 0)),
  )(a, b)


def _pad(x, n):
  return jnp.pad(x.reshape(-1), (0, n - x.shape[0]))


def _bits(x):
  return lax.bitcast_convert_type(x, jnp.int32)


def _pack(n, cols):
  return jnp.stack([c.reshape(n // K, K) for c in cols], axis=1)


def kernel(paper_feat, author_feat, wb_src, wb_dst, wb_eweight,
           pi_src, pi_dst, pi_ealpha, pi_eweight,
           hp_src, hp_dst, hp_feat, hp_ealpha):
  wb_pack = _pack(WB_E, [_pad(wb_src, WB_E), _pad(wb_dst, WB_E),
                         _bits(_pad(wb_eweight, WB_E))])
  pi_pack = _pack(PI_E, [_pad(pi_src, PI_E), _pad(pi_dst, PI_E),
                         _bits(_pad(pi_ealpha, PI_E)),
                         _bits(_pad(pi_eweight, PI_E))])
  hp_pack = _pack(HP_E, [_pad(hp_src, HP_E), _pad(hp_dst, HP_E),
                         _bits(_pad(hp_feat, HP_E)),
                         _bits(_pad(hp_ealpha, HP_E))])

  author_pad, vp0, vp1 = _sc_kernel(
      paper_feat, author_feat, wb_pack, pi_pack, hp_pack)
  venue_h = _combine(vp0, vp1)
  return (paper_feat, author_pad[:N_AUTHOR], venue_h)


# speedup vs baseline: 2.7004x; 1.0532x over previous
import functools

import jax
import jax.numpy as jnp
from jax import lax
from jax.experimental import pallas as pl
from jax.experimental.pallas import tpu as pltpu
from jax.experimental.pallas import tpu_sc as plsc

N_PAPER = 100000
N_AUTHOR = 50000
N_VENUE = 10000
D = 128
L = 16
NS = 16
NC = 2

K = 128
ACHUNK = 8448
NCHUNK = 6
ZB = 40
ACC_ROWS = 10240

WB_BLKS = 2352
PI_BLKS = 800
HP_BLKS = 1568
WB_E = WB_BLKS * K
PI_E = PI_BLKS * K
HP_E = HP_BLKS * K

_mesh = plsc.VectorSubcoreMesh(
    core_axis_name="c", subcore_axis_name="s", num_cores=NC, num_subcores=NS)


def _sc_body(paper, author, wb_pack, pi_pack, hp_pack,
             author_out, vp0, vp1,
             acc, zbuf, rows2, pbufv, pbufa,
             stage_s, stage_d, stage_w, fsrc, fdst, isem, gsem, ssem):
  c = lax.axis_index("c")
  s = lax.axis_index("s")

  @pl.loop(0, ZB)
  def _(r):
    for j in range(D // L):
      zbuf[r, pl.ds(j * L, L)] = jnp.zeros((L,), jnp.float32)

  def zero_acc():
    rows_per_tile = ACC_ROWS // NS
    for i in range(rows_per_tile // ZB):
      pltpu.sync_copy(zbuf, acc.at[pl.ds(s * rows_per_tile + i * ZB, ZB)])
    plsc.subcore_barrier()

  def scale_rows(base, w_from):
    @pl.loop(0, K // L)
    def _(g):
      w16 = w_from(g)
      for lane in range(L):
        r = base + g * L + lane
        w = w16[lane]
        for j in range(D // L):
          sl = pl.ds(j * L, L)
          rows2[r, sl] = rows2[r, sl] * w

  def venue_pass(n_blocks, first_blk, pack_ref, feat_ref):
    def idx_issue(i):
      pltpu.async_copy(pack_ref.at[first_blk + i],
                       pbufv.at[pl.ds(lax.rem(i, 3) * 4, 4)], isem)

    def idx_wait():
      pltpu.make_async_copy(pack_ref.at[0], pbufv.at[pl.ds(0, 4)],
                            isem).wait()

    def gather_issue(i, buf):
      pltpu.async_copy(feat_ref.at[pbufv.at[lax.rem(i, 3) * 4]],
                       rows2.at[pl.ds(buf * K, K)], gsem)

    def gather_wait(buf):
      pltpu.make_async_copy(feat_ref.at[pl.ds(0, K)],
                            rows2.at[pl.ds(buf * K, K)], gsem).wait()

    def scatter_wait():
      pltpu.make_async_copy(feat_ref.at[pl.ds(0, K)],
                            rows2.at[pl.ds(0, K)], ssem).wait()

    idx_issue(0)
    idx_wait()
    gather_issue(0, 0)
    if n_blocks > 1:
      idx_issue(1)

    @pl.loop(0, n_blocks)
    def _(i):
      cur = lax.rem(i, 2)
      slot = lax.rem(i, 3)
      gather_wait(cur)

      @pl.when(i + 1 < n_blocks)
      def _():
        idx_wait()
        pl.when(i >= 1)(scatter_wait)
        gather_issue(i + 1, 1 - cur)

      @pl.when(i + 2 < n_blocks)
      def _():
        idx_issue(i + 2)

      def wv16(g):
        a = plsc.bitcast(pbufv[slot * 4 + 2, pl.ds(g * L, L)], jnp.float32)
        b = plsc.bitcast(pbufv[slot * 4 + 3, pl.ds(g * L, L)], jnp.float32)
        return a * b
      scale_rows(cur * K, wv16)
      pltpu.async_copy(rows2.at[pl.ds(cur * K, K)],
                       acc.at[pbufv.at[slot * 4 + 1]], ssem, add=True)

    for _ in range(min(n_blocks, 2)):
      scatter_wait()

  def flush_stage(feat_ref, not_first):
    @pl.when(not_first)
    def _():
      pltpu.make_async_copy(feat_ref.at[pl.ds(0, K)],
                            rows2.at[pl.ds(0, K)], ssem).wait()
    for g in range(K // L):
      sl = pl.ds(g * L, L)
      fsrc[sl] = stage_s[sl]
      fdst[sl] = stage_d[sl]
    pltpu.sync_copy(feat_ref.at[fsrc], rows2.at[pl.ds(0, K)])
    scale_rows(0, lambda g: stage_w[pl.ds(g * L, L)])
    pltpu.async_copy(rows2.at[pl.ds(0, K)], acc.at[fdst], ssem, add=True)
    for g in range(K // L):
      sl = pl.ds(g * L, L)
      s2 = pl.ds(K + g * L, L)
      stage_s[sl] = stage_s[s2]
      stage_d[sl] = stage_d[s2]
      stage_w[sl] = stage_w[s2]

  def compact_pass(n_blocks, first_blk, pack_ref, feat_ref, chunk_base,
                   trash):
    def idx_issue(i):
      pltpu.async_copy(pack_ref.at[first_blk + i],
                       pbufa.at[pl.ds(lax.rem(i, 2) * 3, 3)], isem)

    def idx_wait():
      pltpu.make_async_copy(pack_ref.at[0], pbufa.at[pl.ds(0, 3)],
                            isem).wait()

    idx_issue(0)

    @pl.loop(0, n_blocks, init_carry=(jnp.int32(0), jnp.int32(0)))
    def carry(b, carry):
      fill, nflush = carry
      slot = lax.rem(b, 2)
      idx_wait()

      @pl.when(b + 1 < n_blocks)
      def _():
        idx_issue(b + 1)

      for g in range(K // L):
        sl = pl.ds(g * L, L)
        s16 = pbufa[slot * 3, sl]
        d16 = pbufa[slot * 3 + 1, sl]
        w16 = plsc.bitcast(pbufa[slot * 3 + 2, sl], jnp.float32)
        msk = (d16 >= chunk_base) & (d16 < chunk_base + ACHUNK)
        sdst, ssrc, _ = plsc.sort_key_val(d16 - chunk_base, s16, mask=msk)
        _, sw, _ = plsc.sort_key_val(d16 - chunk_base, w16, mask=msk)
        stage_s[pl.ds(fill, L)] = ssrc
        stage_d[pl.ds(fill, L)] = sdst
        stage_w[pl.ds(fill, L)] = sw
        cnt = plsc.all_reduce_population_count(msk)
        if getattr(cnt, "ndim", 0):
          cnt = cnt[0]
        fill = fill + cnt
      do = fill >= K
      pl.when(do)(lambda: flush_stage(feat_ref, nflush > 0))
      return (jnp.where(do, fill - K, fill),
              jnp.where(do, nflush + 1, nflush))

    fill, nflush = carry
    for g in range(K // L):
      sl = pl.ds(g * L, L)
      pos = lax.iota(jnp.int32, L) + g * L
      valid = pos < fill
      stage_d[sl] = jnp.where(valid, stage_d[sl], trash)
      stage_s[sl] = jnp.where(valid, stage_s[sl], 0)
    flush_stage(feat_ref, nflush > 0)
    pltpu.make_async_copy(feat_ref.at[pl.ds(0, K)],
                          rows2.at[pl.ds(0, K)], ssem).wait()

  vrows = 1000
  zero_acc()
  venue_pass(PI_BLKS // (NC * NS),
             c * (PI_BLKS // NC) + s * (PI_BLKS // (NC * NS)), pi_pack, paper)
  venue_pass(HP_BLKS // (NC * NS),
             c * (HP_BLKS // NC) + s * (HP_BLKS // (NC * NS)), hp_pack,
             author)
  plsc.subcore_barrier()
  vb = s * vrows

  @pl.when((c == 0) & (s < N_VENUE // vrows))
  def _():
    pltpu.sync_copy(acc.at[pl.ds(vb, vrows)], vp0.at[pl.ds(vb, vrows)])

  @pl.when((c == 1) & (s < N_VENUE // vrows))
  def _():
    pltpu.sync_copy(acc.at[pl.ds(vb, vrows)], vp1.at[pl.ds(vb, vrows)])
  plsc.subcore_barrier()

  per_tile = ACHUNK // NS
  for chunk in range(NCHUNK):
    owner = chunk // 3
    crow = chunk * ACHUNK

    @pl.when(c == owner)
    def _():
      zero_acc()
      compact_pass(WB_BLKS // NS, s * (WB_BLKS // NS),
                   wb_pack, paper, crow, ACHUNK)
      plsc.subcore_barrier()
      wbase = s * per_tile
      pltpu.sync_copy(acc.at[pl.ds(wbase, per_tile)],
                      author_out.at[pl.ds(crow + wbase, per_tile)])
      plsc.subcore_barrier()


_sc_kernel = functools.partial(
    pl.kernel,
    out_type=[
        jax.ShapeDtypeStruct((NCHUNK * ACHUNK, D), jnp.float32),
        jax.ShapeDtypeStruct((N_VENUE, D), jnp.float32),
        jax.ShapeDtypeStruct((N_VENUE, D), jnp.float32),
    ],
    mesh=_mesh,
    compiler_params=pltpu.CompilerParams(needs_layout_passes=False),
    scratch_types=[
        pltpu.VMEM_SHARED((ACC_ROWS, D), jnp.float32),
        pltpu.VMEM((ZB, D), jnp.float32),
        pltpu.VMEM((2 * K, D), jnp.float32),
        pltpu.VMEM((12, K), jnp.int32),
        pltpu.VMEM((6, K), jnp.int32),
        pltpu.VMEM((2 * K,), jnp.int32),
        pltpu.VMEM((2 * K,), jnp.int32),
        pltpu.VMEM((2 * K,), jnp.float32),
        pltpu.VMEM((K,), jnp.int32),
        pltpu.VMEM((K,), jnp.int32),
        pltpu.SemaphoreType.DMA,
        pltpu.SemaphoreType.DMA,
        pltpu.SemaphoreType.DMA,
    ],
)(_sc_body)


def _add_body(a_ref, b_ref, o_ref):
  o_ref[...] = a_ref[...] + b_ref[...]


def _combine(a, b):
  return pl.pallas_call(
      _add_body,
      out_shape=jax.ShapeDtypeStruct((N_VENUE, D), jnp.float32),
      grid=(10,),
      in_specs=[pl.BlockSpec((N_VENUE // 10, D), lambda i: (i, 0))] * 2,
      out_specs=pl.BlockSpec((N_VENUE // 10, D), lambda i: (i, 0)),
  )(a, b)


def _pad(x, n):
  return jnp.pad(x.reshape(-1), (0, n - x.shape[0]))


def _bits(x):
  return lax.bitcast_convert_type(x, jnp.int32)


def _pack(n, cols):
  return jnp.stack([c.reshape(n // K, K) for c in cols], axis=1)


def kernel(paper_feat, author_feat, wb_src, wb_dst, wb_eweight,
           pi_src, pi_dst, pi_ealpha, pi_eweight,
           hp_src, hp_dst, hp_feat, hp_ealpha):
  wb_pack = _pack(WB_E, [_pad(wb_src, WB_E), _pad(wb_dst, WB_E),
                         _bits(_pad(wb_eweight, WB_E))])
  pi_pack = _pack(PI_E, [_pad(pi_src, PI_E), _pad(pi_dst, PI_E),
                         _bits(_pad(pi_ealpha, PI_E)),
                         _bits(_pad(pi_eweight, PI_E))])
  hp_pack = _pack(HP_E, [_pad(hp_src, HP_E), _pad(hp_dst, HP_E),
                         _bits(_pad(hp_feat, HP_E)),
                         _bits(_pad(hp_ealpha, HP_E))])

  author_pad, vp0, vp1 = _sc_kernel(
      paper_feat, author_feat, wb_pack, pi_pack, hp_pack)
  venue_h = _combine(vp0, vp1)
  return (paper_feat, author_pad[:N_AUTHOR], venue_h)


# speedup vs baseline: 3.4189x; 1.2661x over previous
import functools

import jax
import jax.numpy as jnp
from jax import lax
from jax.experimental import pallas as pl
from jax.experimental.pallas import tpu as pltpu
from jax.experimental.pallas import tpu_sc as plsc

N_PAPER = 100000
N_AUTHOR = 50000
N_VENUE = 10000
D = 128
L = 16
NS = 16
NC = 2

K = 128
ACHUNK = 8448
NCHUNK = 6
ZB = 40
ACC_ROWS = 10240

WB_BLKS = 2352
PI_BLKS = 800
HP_BLKS = 1568
WB_E = WB_BLKS * K
PI_E = PI_BLKS * K
HP_E = HP_BLKS * K

_mesh = plsc.VectorSubcoreMesh(
    core_axis_name="c", subcore_axis_name="s", num_cores=NC, num_subcores=NS)


def _sc_body(paper, author, wb_pack, pi_pack, hp_pack,
             author_out, vp0, vp1,
             acc, zbuf, rows2, pbufv, pbufa,
             stage_s, stage_d, stage_w, fsrc, fdst, isem, gsem, ssem):
  c = lax.axis_index("c")
  s = lax.axis_index("s")

  @pl.loop(0, ZB)
  def _(r):
    for j in range(D // L):
      zbuf[r, pl.ds(j * L, L)] = jnp.zeros((L,), jnp.float32)

  def zero_acc():
    rows_per_tile = ACC_ROWS // NS
    for i in range(rows_per_tile // ZB):
      pltpu.sync_copy(zbuf, acc.at[pl.ds(s * rows_per_tile + i * ZB, ZB)])
    plsc.subcore_barrier()

  def scale_rows(base, w_from):
    @plsc.parallel_loop(0, K // L, unroll=2)
    def _(g):
      w16 = w_from(g)
      for lane in range(L):
        r = base + g * L + lane
        w = w16[lane]
        for j in range(D // L):
          sl = pl.ds(j * L, L)
          rows2[r, sl] = rows2[r, sl] * w

  def venue_pass(n_blocks, first_blk, pack_ref, feat_ref):
    def idx_issue(i):
      pltpu.async_copy(pack_ref.at[first_blk + i],
                       pbufv.at[pl.ds(lax.rem(i, 3) * 4, 4)], isem)

    def idx_wait():
      pltpu.make_async_copy(pack_ref.at[0], pbufv.at[pl.ds(0, 4)],
                            isem).wait()

    def gather_issue(i, buf):
      pltpu.async_copy(feat_ref.at[pbufv.at[lax.rem(i, 3) * 4]],
                       rows2.at[pl.ds(buf * K, K)], gsem)

    def gather_wait(buf):
      pltpu.make_async_copy(feat_ref.at[pl.ds(0, K)],
                            rows2.at[pl.ds(buf * K, K)], gsem).wait()

    def scatter_wait():
      pltpu.make_async_copy(feat_ref.at[pl.ds(0, K)],
                            rows2.at[pl.ds(0, K)], ssem).wait()

    idx_issue(0)
    idx_wait()
    gather_issue(0, 0)
    if n_blocks > 1:
      idx_issue(1)

    @pl.loop(0, n_blocks)
    def _(i):
      cur = lax.rem(i, 2)
      slot = lax.rem(i, 3)
      gather_wait(cur)

      @pl.when(i + 1 < n_blocks)
      def _():
        idx_wait()
        pl.when(i >= 1)(scatter_wait)
        gather_issue(i + 1, 1 - cur)

      @pl.when(i + 2 < n_blocks)
      def _():
        idx_issue(i + 2)

      def wv16(g):
        a = plsc.bitcast(pbufv[slot * 4 + 2, pl.ds(g * L, L)], jnp.float32)
        b = plsc.bitcast(pbufv[slot * 4 + 3, pl.ds(g * L, L)], jnp.float32)
        return a * b
      scale_rows(cur * K, wv16)
      pltpu.async_copy(rows2.at[pl.ds(cur * K, K)],
                       acc.at[pbufv.at[slot * 4 + 1]], ssem, add=True)

    for _ in range(min(n_blocks, 2)):
      scatter_wait()

  def flush_stage(feat_ref, not_first):
    @pl.when(not_first)
    def _():
      pltpu.make_async_copy(feat_ref.at[pl.ds(0, K)],
                            rows2.at[pl.ds(0, K)], ssem).wait()
    for g in range(K // L):
      sl = pl.ds(g * L, L)
      fsrc[sl] = stage_s[sl]
      fdst[sl] = stage_d[sl]
    pltpu.sync_copy(feat_ref.at[fsrc], rows2.at[pl.ds(0, K)])
    scale_rows(0, lambda g: stage_w[pl.ds(g * L, L)])
    pltpu.async_copy(rows2.at[pl.ds(0, K)], acc.at[fdst], ssem, add=True)
    for g in range(K // L):
      sl = pl.ds(g * L, L)
      s2 = pl.ds(K + g * L, L)
      stage_s[sl] = stage_s[s2]
      stage_d[sl] = stage_d[s2]
      stage_w[sl] = stage_w[s2]

  def compact_pass(n_blocks, first_blk, pack_ref, feat_ref, chunk_base,
                   trash):
    def idx_issue(i):
      pltpu.async_copy(pack_ref.at[first_blk + i],
                       pbufa.at[pl.ds(lax.rem(i, 2) * 3, 3)], isem)

    def idx_wait():
      pltpu.make_async_copy(pack_ref.at[0], pbufa.at[pl.ds(0, 3)],
                            isem).wait()

    idx_issue(0)

    @pl.loop(0, n_blocks, init_carry=(jnp.int32(0), jnp.int32(0)))
    def carry(b, carry):
      fill, nflush = carry
      slot = lax.rem(b, 2)
      idx_wait()

      @pl.when(b + 1 < n_blocks)
      def _():
        idx_issue(b + 1)

      for g in range(K // L):
        sl = pl.ds(g * L, L)
        s16 = pbufa[slot * 3, sl]
        d16 = pbufa[slot * 3 + 1, sl]
        w16 = plsc.bitcast(pbufa[slot * 3 + 2, sl], jnp.float32)
        msk = (d16 >= chunk_base) & (d16 < chunk_base + ACHUNK)
        sdst, ssrc, _ = plsc.sort_key_val(d16 - chunk_base, s16, mask=msk)
        _, sw, _ = plsc.sort_key_val(d16 - chunk_base, w16, mask=msk)
        stage_s[pl.ds(fill, L)] = ssrc
        stage_d[pl.ds(fill, L)] = sdst
        stage_w[pl.ds(fill, L)] = sw
        cnt = plsc.all_reduce_population_count(msk)
        if getattr(cnt, "ndim", 0):
          cnt = cnt[0]
        fill = fill + cnt
      do = fill >= K
      pl.when(do)(lambda: flush_stage(feat_ref, nflush > 0))
      return (jnp.where(do, fill - K, fill),
              jnp.where(do, nflush + 1, nflush))

    fill, nflush = carry
    for g in range(K // L):
      sl = pl.ds(g * L, L)
      pos = lax.iota(jnp.int32, L) + g * L
      valid = pos < fill
      stage_d[sl] = jnp.where(valid, stage_d[sl], trash)
      stage_s[sl] = jnp.where(valid, stage_s[sl], 0)
    flush_stage(feat_ref, nflush > 0)
    pltpu.make_async_copy(feat_ref.at[pl.ds(0, K)],
                          rows2.at[pl.ds(0, K)], ssem).wait()

  vrows = 1000
  zero_acc()
  venue_pass(PI_BLKS // (NC * NS),
             c * (PI_BLKS // NC) + s * (PI_BLKS // (NC * NS)), pi_pack, paper)
  venue_pass(HP_BLKS // (NC * NS),
             c * (HP_BLKS // NC) + s * (HP_BLKS // (NC * NS)), hp_pack,
             author)
  plsc.subcore_barrier()
  vb = s * vrows

  @pl.when((c == 0) & (s < N_VENUE // vrows))
  def _():
    pltpu.sync_copy(acc.at[pl.ds(vb, vrows)], vp0.at[pl.ds(vb, vrows)])

  @pl.when((c == 1) & (s < N_VENUE // vrows))
  def _():
    pltpu.sync_copy(acc.at[pl.ds(vb, vrows)], vp1.at[pl.ds(vb, vrows)])
  plsc.subcore_barrier()

  per_tile = ACHUNK // NS
  for chunk in range(NCHUNK):
    owner = chunk // 3
    crow = chunk * ACHUNK

    @pl.when(c == owner)
    def _():
      zero_acc()
      compact_pass(WB_BLKS // NS, s * (WB_BLKS // NS),
                   wb_pack, paper, crow, ACHUNK)
      plsc.subcore_barrier()
      wbase = s * per_tile
      pltpu.sync_copy(acc.at[pl.ds(wbase, per_tile)],
                      author_out.at[pl.ds(crow + wbase, per_tile)])
      plsc.subcore_barrier()


_sc_kernel = functools.partial(
    pl.kernel,
    out_type=[
        jax.ShapeDtypeStruct((NCHUNK * ACHUNK, D), jnp.float32),
        jax.ShapeDtypeStruct((N_VENUE, D), jnp.float32),
        jax.ShapeDtypeStruct((N_VENUE, D), jnp.float32),
    ],
    mesh=_mesh,
    compiler_params=pltpu.CompilerParams(needs_layout_passes=False),
    scratch_types=[
        pltpu.VMEM_SHARED((ACC_ROWS, D), jnp.float32),
        pltpu.VMEM((ZB, D), jnp.float32),
        pltpu.VMEM((2 * K, D), jnp.float32),
        pltpu.VMEM((12, K), jnp.int32),
        pltpu.VMEM((6, K), jnp.int32),
        pltpu.VMEM((2 * K,), jnp.int32),
        pltpu.VMEM((2 * K,), jnp.int32),
        pltpu.VMEM((2 * K,), jnp.float32),
        pltpu.VMEM((K,), jnp.int32),
        pltpu.VMEM((K,), jnp.int32),
        pltpu.SemaphoreType.DMA,
        pltpu.SemaphoreType.DMA,
        pltpu.SemaphoreType.DMA,
    ],
)(_sc_body)


def _add_body(a_ref, b_ref, o_ref):
  o_ref[...] = a_ref[...] + b_ref[...]


def _combine(a, b):
  return pl.pallas_call(
      _add_body,
      out_shape=jax.ShapeDtypeStruct((N_VENUE, D), jnp.float32),
      grid=(10,),
      in_specs=[pl.BlockSpec((N_VENUE // 10, D), lambda i: (i, 0))] * 2,
      out_specs=pl.BlockSpec((N_VENUE // 10, D), lambda i: (i, 0)),
  )(a, b)


def _pad(x, n):
  return jnp.pad(x.reshape(-1), (0, n - x.shape[0]))


def _bits(x):
  return lax.bitcast_convert_type(x, jnp.int32)


def _pack(n, cols):
  return jnp.stack([c.reshape(n // K, K) for c in cols], axis=1)


def kernel(paper_feat, author_feat, wb_src, wb_dst, wb_eweight,
           pi_src, pi_dst, pi_ealpha, pi_eweight,
           hp_src, hp_dst, hp_feat, hp_ealpha):
  wb_pack = _pack(WB_E, [_pad(wb_src, WB_E), _pad(wb_dst, WB_E),
                         _bits(_pad(wb_eweight, WB_E))])
  pi_pack = _pack(PI_E, [_pad(pi_src, PI_E), _pad(pi_dst, PI_E),
                         _bits(_pad(pi_ealpha, PI_E)),
                         _bits(_pad(pi_eweight, PI_E))])
  hp_pack = _pack(HP_E, [_pad(hp_src, HP_E), _pad(hp_dst, HP_E),
                         _bits(_pad(hp_feat, HP_E)),
                         _bits(_pad(hp_ealpha, HP_E))])

  author_pad, vp0, vp1 = _sc_kernel(
      paper_feat, author_feat, wb_pack, pi_pack, hp_pack)
  venue_h = _combine(vp0, vp1)
  return (paper_feat, author_pad[:N_AUTHOR], venue_h)


# speedup vs baseline: 3.4482x; 1.0086x over previous
import functools

import jax
import jax.numpy as jnp
from jax import lax
from jax.experimental import pallas as pl
from jax.experimental.pallas import tpu as pltpu
from jax.experimental.pallas import tpu_sc as plsc

N_PAPER = 100000
N_AUTHOR = 50000
N_VENUE = 10000
D = 128
L = 16
NS = 16
NC = 2

K = 128
ACHUNK = 8448
NCHUNK = 6
ZB = 40
ACC_ROWS = 10240

WB_BLKS = 2352
PI_BLKS = 800
HP_BLKS = 1568
WB_E = WB_BLKS * K
PI_E = PI_BLKS * K
HP_E = HP_BLKS * K

_mesh = plsc.VectorSubcoreMesh(
    core_axis_name="c", subcore_axis_name="s", num_cores=NC, num_subcores=NS)


def _sc_body(paper, author, wb_pack, pi_pack, hp_pack,
             author_out, vp0, vp1,
             acc, zbuf, rows2, pbufv, pbufa,
             stage_s, stage_d, stage_w, fsrc, fdst, fw, isem, gsem, ssem):
  c = lax.axis_index("c")
  s = lax.axis_index("s")

  @pl.loop(0, ZB)
  def _(r):
    for j in range(D // L):
      zbuf[r, pl.ds(j * L, L)] = jnp.zeros((L,), jnp.float32)

  def zero_acc():
    rows_per_tile = ACC_ROWS // NS
    for i in range(rows_per_tile // ZB):
      pltpu.sync_copy(zbuf, acc.at[pl.ds(s * rows_per_tile + i * ZB, ZB)])
    plsc.subcore_barrier()

  def scale_rows(base, w_from):
    @plsc.parallel_loop(0, K // L, unroll=2)
    def _(g):
      w16 = w_from(g)
      for lane in range(L):
        r = base + g * L + lane
        w = w16[lane]
        for j in range(D // L):
          sl = pl.ds(j * L, L)
          rows2[r, sl] = rows2[r, sl] * w

  def venue_pass(n_blocks, first_blk, pack_ref, feat_ref):
    def idx_issue(i):
      pltpu.async_copy(pack_ref.at[first_blk + i],
                       pbufv.at[pl.ds(lax.rem(i, 3) * 4, 4)], isem)

    def idx_wait():
      pltpu.make_async_copy(pack_ref.at[0], pbufv.at[pl.ds(0, 4)],
                            isem).wait()

    def gather_issue(i, buf):
      pltpu.async_copy(feat_ref.at[pbufv.at[lax.rem(i, 3) * 4]],
                       rows2.at[pl.ds(buf * K, K)], gsem)

    def gather_wait(buf):
      pltpu.make_async_copy(feat_ref.at[pl.ds(0, K)],
                            rows2.at[pl.ds(buf * K, K)], gsem).wait()

    def scatter_wait():
      pltpu.make_async_copy(feat_ref.at[pl.ds(0, K)],
                            rows2.at[pl.ds(0, K)], ssem).wait()

    idx_issue(0)
    idx_wait()
    gather_issue(0, 0)
    if n_blocks > 1:
      idx_issue(1)

    @pl.loop(0, n_blocks)
    def _(i):
      cur = lax.rem(i, 2)
      slot = lax.rem(i, 3)
      gather_wait(cur)

      @pl.when(i + 1 < n_blocks)
      def _():
        idx_wait()
        pl.when(i >= 1)(scatter_wait)
        gather_issue(i + 1, 1 - cur)

      @pl.when(i + 2 < n_blocks)
      def _():
        idx_issue(i + 2)

      def wv16(g):
        a = plsc.bitcast(pbufv[slot * 4 + 2, pl.ds(g * L, L)], jnp.float32)
        b = plsc.bitcast(pbufv[slot * 4 + 3, pl.ds(g * L, L)], jnp.float32)
        return a * b
      scale_rows(cur * K, wv16)
      pltpu.async_copy(rows2.at[pl.ds(cur * K, K)],
                       acc.at[pbufv.at[slot * 4 + 1]], ssem, add=True)

    for _ in range(min(n_blocks, 2)):
      scatter_wait()

  def sc_wait(feat_ref):
    pltpu.make_async_copy(feat_ref.at[pl.ds(0, K)],
                          rows2.at[pl.ds(0, K)], ssem).wait()

  def g_wait(feat_ref):
    pltpu.make_async_copy(feat_ref.at[pl.ds(0, K)],
                          rows2.at[pl.ds(0, K)], gsem).wait()

  def flush_process(feat_ref, q):
    g_wait(feat_ref)
    scale_rows(q * K, lambda g: fw[q, pl.ds(g * L, L)])
    pltpu.async_copy(rows2.at[pl.ds(q * K, K)], acc.at[fdst.at[q]], ssem,
                     add=True)

  def flush_stage(feat_ref, nflush):
    pp = lax.rem(nflush, 2)

    @pl.when(nflush >= 2)
    def _():
      sc_wait(feat_ref)
    for g in range(K // L):
      sl = pl.ds(g * L, L)
      fsrc[pp, sl] = stage_s[sl]
      fdst[pp, sl] = stage_d[sl]
      fw[pp, sl] = stage_w[sl]
    pltpu.async_copy(feat_ref.at[fsrc.at[pp]], rows2.at[pl.ds(pp * K, K)],
                     gsem)
    pl.when(nflush >= 1)(lambda: flush_process(feat_ref, 1 - pp))
    for g in range(K // L):
      sl = pl.ds(g * L, L)
      s2 = pl.ds(K + g * L, L)
      stage_s[sl] = stage_s[s2]
      stage_d[sl] = stage_d[s2]
      stage_w[sl] = stage_w[s2]

  def compact_pass(n_blocks, first_blk, pack_ref, feat_ref, chunk_base,
                   trash):
    def idx_issue(i):
      pltpu.async_copy(pack_ref.at[first_blk + i],
                       pbufa.at[pl.ds(lax.rem(i, 2) * 3, 3)], isem)

    def idx_wait():
      pltpu.make_async_copy(pack_ref.at[0], pbufa.at[pl.ds(0, 3)],
                            isem).wait()

    idx_issue(0)

    @pl.loop(0, n_blocks, init_carry=(jnp.int32(0), jnp.int32(0)))
    def carry(b, carry):
      fill, nflush = carry
      slot = lax.rem(b, 2)
      idx_wait()

      @pl.when(b + 1 < n_blocks)
      def _():
        idx_issue(b + 1)

      for g in range(K // L):
        sl = pl.ds(g * L, L)
        s16 = pbufa[slot * 3, sl]
        d16 = pbufa[slot * 3 + 1, sl]
        w16 = plsc.bitcast(pbufa[slot * 3 + 2, sl], jnp.float32)
        msk = (d16 >= chunk_base) & (d16 < chunk_base + ACHUNK)
        sdst, ssrc, _ = plsc.sort_key_val(d16 - chunk_base, s16, mask=msk)
        _, sw, _ = plsc.sort_key_val(d16 - chunk_base, w16, mask=msk)
        stage_s[pl.ds(fill, L)] = ssrc
        stage_d[pl.ds(fill, L)] = sdst
        stage_w[pl.ds(fill, L)] = sw
        cnt = plsc.all_reduce_population_count(msk)
        if getattr(cnt, "ndim", 0):
          cnt = cnt[0]
        fill = fill + cnt
      do = fill >= K
      pl.when(do)(lambda: flush_stage(feat_ref, nflush))
      return (jnp.where(do, fill - K, fill),
              jnp.where(do, nflush + 1, nflush))

    fill, nflush = carry
    for g in range(K // L):
      sl = pl.ds(g * L, L)
      pos = lax.iota(jnp.int32, L) + g * L
      valid = pos < fill
      stage_d[sl] = jnp.where(valid, stage_d[sl], trash)
      stage_s[sl] = jnp.where(valid, stage_s[sl], 0)
    flush_stage(feat_ref, nflush)
    flush_process(feat_ref, lax.rem(nflush, 2))
    pl.when(nflush >= 1)(lambda: sc_wait(feat_ref))
    sc_wait(feat_ref)

  vrows = 1000
  zero_acc()
  venue_pass(PI_BLKS // (NC * NS),
             c * (PI_BLKS // NC) + s * (PI_BLKS // (NC * NS)), pi_pack, paper)
  venue_pass(HP_BLKS // (NC * NS),
             c * (HP_BLKS // NC) + s * (HP_BLKS // (NC * NS)), hp_pack,
             author)
  plsc.subcore_barrier()
  vb = s * vrows

  @pl.when((c == 0) & (s < N_VENUE // vrows))
  def _():
    pltpu.sync_copy(acc.at[pl.ds(vb, vrows)], vp0.at[pl.ds(vb, vrows)])

  @pl.when((c == 1) & (s < N_VENUE // vrows))
  def _():
    pltpu.sync_copy(acc.at[pl.ds(vb, vrows)], vp1.at[pl.ds(vb, vrows)])
  plsc.subcore_barrier()

  per_tile = ACHUNK // NS

  @pl.loop(0, NCHUNK)
  def _(chunk):
    owner = chunk // 3
    crow = pl.multiple_of(chunk * ACHUNK, 8)

    @pl.when(c == owner)
    def _():
      zero_acc()
      compact_pass(WB_BLKS // NS, s * (WB_BLKS // NS),
                   wb_pack, paper, crow, ACHUNK)
      plsc.subcore_barrier()
      wbase = s * per_tile
      pltpu.sync_copy(acc.at[pl.ds(wbase, per_tile)],
                      author_out.at[pl.ds(crow + wbase, per_tile)])
      plsc.subcore_barrier()


_sc_kernel = functools.partial(
    pl.kernel,
    out_type=[
        jax.ShapeDtypeStruct((NCHUNK * ACHUNK, D), jnp.float32),
        jax.ShapeDtypeStruct((N_VENUE, D), jnp.float32),
        jax.ShapeDtypeStruct((N_VENUE, D), jnp.float32),
    ],
    mesh=_mesh,
    compiler_params=pltpu.CompilerParams(needs_layout_passes=False),
    scratch_types=[
        pltpu.VMEM_SHARED((ACC_ROWS, D), jnp.float32),
        pltpu.VMEM((ZB, D), jnp.float32),
        pltpu.VMEM((2 * K, D), jnp.float32),
        pltpu.VMEM((12, K), jnp.int32),
        pltpu.VMEM((6, K), jnp.int32),
        pltpu.VMEM((2 * K,), jnp.int32),
        pltpu.VMEM((2 * K,), jnp.int32),
        pltpu.VMEM((2 * K,), jnp.float32),
        pltpu.VMEM((2, K), jnp.int32),
        pltpu.VMEM((2, K), jnp.int32),
        pltpu.VMEM((2, K), jnp.float32),
        pltpu.SemaphoreType.DMA,
        pltpu.SemaphoreType.DMA,
        pltpu.SemaphoreType.DMA,
    ],
)(_sc_body)


def _add_body(a_ref, b_ref, o_ref):
  o_ref[...] = a_ref[...] + b_ref[...]


def _combine(a, b):
  return pl.pallas_call(
      _add_body,
      out_shape=jax.ShapeDtypeStruct((N_VENUE, D), jnp.float32),
      grid=(10,),
      in_specs=[pl.BlockSpec((N_VENUE // 10, D), lambda i: (i, 0))] * 2,
      out_specs=pl.BlockSpec((N_VENUE // 10, D), lambda i: (i, 0)),
  )(a, b)


def _pad(x, n):
  return jnp.pad(x.reshape(-1), (0, n - x.shape[0]))


def _bits(x):
  return lax.bitcast_convert_type(x, jnp.int32)


def _pack(n, cols):
  return jnp.stack([c.reshape(n // K, K) for c in cols], axis=1)


def kernel(paper_feat, author_feat, wb_src, wb_dst, wb_eweight,
           pi_src, pi_dst, pi_ealpha, pi_eweight,
           hp_src, hp_dst, hp_feat, hp_ealpha):
  wb_pack = _pack(WB_E, [_pad(wb_src, WB_E), _pad(wb_dst, WB_E),
                         _bits(_pad(wb_eweight, WB_E))])
  pi_pack = _pack(PI_E, [_pad(pi_src, PI_E), _pad(pi_dst, PI_E),
                         _bits(_pad(pi_ealpha, PI_E)),
                         _bits(_pad(pi_eweight, PI_E))])
  hp_pack = _pack(HP_E, [_pad(hp_src, HP_E), _pad(hp_dst, HP_E),
                         _bits(_pad(hp_feat, HP_E)),
                         _bits(_pad(hp_ealpha, HP_E))])

  author_pad, vp0, vp1 = _sc_kernel(
      paper_feat, author_feat, wb_pack, pi_pack, hp_pack)
  venue_h = _combine(vp0, vp1)
  return (paper_feat, author_pad[:N_AUTHOR], venue_h)
